# baseline (speedup 1.0000x reference)
import functools

import jax
import jax.numpy as jnp
from jax import lax
from jax.experimental import pallas as pl
from jax.experimental.pallas import tpu as pltpu
from jax.experimental.pallas import tpu_sc as plsc


_NBUF_TC = 8


def _tc_pool_body(x_hbm, o_ref, vmem, sem, *, nslab, slab_rows):
    def start(si):
        slot = lax.rem(si, _NBUF_TC)
        pltpu.make_async_copy(
            x_hbm.at[pl.ds(si * slab_rows, slab_rows)],
            vmem.at[slot],
            sem.at[slot],
        ).start()

    for s in range(_NBUF_TC - 1):
        start(s)

    def step(si, _):
        slot = lax.rem(si, _NBUF_TC)
        nxt = si + _NBUF_TC - 1

        @pl.when(nxt < nslab)
        def _():
            start(nxt)

        pltpu.make_async_copy(
            x_hbm.at[pl.ds(si * slab_rows, slab_rows)],
            vmem.at[slot],
            sem.at[slot],
        ).wait()
        o_ref[pl.ds(si * slab_rows, slab_rows)] = jnp.sum(
            vmem[slot], axis=(1, 2), keepdims=True
        )
        return 0

    lax.fori_loop(0, nslab, step, 0)


_NC = 2
_NS = 16
_NW = _NC * _NS
_L = 16
_NBUF_SC = 3
_AHEAD = 2
_HCHUNKS = 6


def _sc_pool_body(x_hbm, out_hbm, *refs, plane_base, planes_per_tile,
                  hrows, width):
    bufs = refs[:_NBUF_SC]
    prow = refs[_NBUF_SC]
    sems = refs[_NBUF_SC + 1:]
    wid = lax.axis_index("s") * _NC + lax.axis_index("c")
    plane0 = plane_base + wid * planes_per_tile
    vecs_per_hrow = width // _L

    def start(r, hc, slot):
        pltpu.async_copy(
            x_hbm.at[plane0 + r, pl.ds(hc * hrows, hrows), :],
            bufs[slot],
            sems[slot],
        )

    for g in range(_AHEAD):
        start(0, g, g % _NBUF_SC)

    @pl.loop(0, planes_per_tile)
    def _planes(r):
        accs = [jnp.zeros((_L,), jnp.float32)] * 8
        for hc in range(_HCHUNKS):
            slot = hc % _NBUF_SC
            nslot = (hc + _AHEAD) % _NBUF_SC

            if hc + _AHEAD < _HCHUNKS:

                @pl.when(r < planes_per_tile)
                def _start_same_plane():
                    start(r, hc + _AHEAD, nslot)

            else:

                @pl.when(r + 1 < planes_per_tile)
                def _start_next_plane():
                    start(r + 1, hc + _AHEAD - _HCHUNKS, nslot)

            buf = bufs[slot]
            pltpu.make_async_copy(
                x_hbm.at[plane0, pl.ds(0, hrows), :], buf, sems[slot]
            ).wait()

            @pl.loop(0, hrows, init_carry=tuple(accs))
            def _acc(i, acc):
                acc = list(acc)
                for k in range(vecs_per_hrow):
                    acc[k % 8] = acc[k % 8] + buf[i, pl.ds(k * _L, _L)]
                return tuple(acc)

            accs = list(_acc)

        a = accs
        v = ((a[0] + a[1]) + (a[2] + a[3])) + \
            ((a[4] + a[5]) + (a[6] + a[7]))
        prow[pl.ds(pl.multiple_of(r * _L, _L), _L)] = v

    pltpu.sync_copy(prow, out_hbm.at[pl.ds(wid * planes_per_tile * _L,
                                           planes_per_tile * _L)])


def _proj_body(ptc_ref, psc_ref, mtc_ref, msc_ref, o_ref, *, inv_n):
    s_tc = ptc_ref[...] * inv_n
    s_sc = jnp.sum(psc_ref[...], axis=1, keepdims=True) * inv_n
    o_ref[...] = jax.lax.dot_general(
        mtc_ref[...], s_tc, (((1,), (0,)), ((), ())),
        preferred_element_type=jnp.float32,
    ) + jax.lax.dot_general(
        msc_ref[...], s_sc, (((1,), (0,)), ((), ())),
        preferred_element_type=jnp.float32,
    )


def kernel(x, W):
    B, C, H, Wd = x.shape
    N = H * Wd
    E = W.shape[0]
    R = B * C

    r_sc = 352
    r_tc = R - r_sc

    slab_rows = 8
    nslab = r_tc // slab_rows

    x3 = x.reshape(R, H, Wd)

    tc_sums = pl.pallas_call(
        functools.partial(_tc_pool_body, nslab=nslab, slab_rows=slab_rows),
        in_specs=[pl.BlockSpec(memory_space=pl.ANY)],
        out_specs=pl.BlockSpec(memory_space=pltpu.MemorySpace.VMEM),
        out_shape=jax.ShapeDtypeStruct((r_tc, 1, 1), jnp.float32),
        scratch_shapes=[
            pltpu.VMEM((_NBUF_TC, slab_rows, H, Wd), jnp.float32),
            pltpu.SemaphoreType.DMA((_NBUF_TC,)),
        ],
        compiler_params=pltpu.CompilerParams(
            vmem_limit_bytes=100 * 1024 * 1024,
        ),
    )(x3)

    planes_per_tile = r_sc // _NW
    hrows = H // _HCHUNKS

    mesh = plsc.VectorSubcoreMesh(core_axis_name="c", subcore_axis_name="s")
    sc_partials = pl.kernel(
        functools.partial(
            _sc_pool_body,
            plane_base=r_tc,
            planes_per_tile=planes_per_tile,
            hrows=hrows,
            width=Wd,
        ),
        out_type=jax.ShapeDtypeStruct((r_sc * _L,), jnp.float32),
        mesh=mesh,
        scratch_types=(
            [pltpu.VMEM((hrows, Wd), jnp.float32)] * _NBUF_SC
            + [pltpu.VMEM((planes_per_tile * _L,), jnp.float32)]
            + [pltpu.SemaphoreType.DMA] * _NBUF_SC
        ),
    )(x3)

    M = (jnp.eye(B, dtype=jnp.float32)[:, None, :, None]
         * W[None, :, None, :]).reshape(B * E, R)

    logits_flat = pl.pallas_call(
        functools.partial(_proj_body, inv_n=1.0 / N),
        in_specs=[
            pl.BlockSpec((r_tc, 1), lambda: (0, 0)),
            pl.BlockSpec((r_sc, _L), lambda: (0, 0)),
            pl.BlockSpec((B * E, r_tc), lambda: (0, 0)),
            pl.BlockSpec((B * E, r_sc), lambda: (0, 0)),
        ],
        out_specs=pl.BlockSpec((B * E, 1), lambda: (0, 0)),
        out_shape=jax.ShapeDtypeStruct((B * E, 1), jnp.float32),
    )(tc_sums.reshape(r_tc, 1), sc_partials.reshape(r_sc, _L),
      M[:, :r_tc], M[:, r_tc:])

    return logits_flat.reshape(B, E)


# speedup vs baseline: 1.0254x; 1.0254x over previous
import functools

import jax
import jax.numpy as jnp
from jax import lax
from jax.experimental import pallas as pl
from jax.experimental.pallas import tpu as pltpu
from jax.experimental.pallas import tpu_sc as plsc


_NBUF_TC = 8


def _tc_pool_body(x_hbm, o_ref, vmem, sem, *, nslab, slab_rows):
    def start(si, prio):
        slot = lax.rem(si, _NBUF_TC)
        pltpu.async_copy(
            x_hbm.at[pl.ds(si * slab_rows, slab_rows)],
            vmem.at[slot],
            sem.at[slot],
            priority=prio,
        )

    for s in range(_NBUF_TC - 1):
        start(s, s % 2)

    @pl.loop(0, nslab, step=2)
    def _steps(g):
        for u in range(2):
            si = g + u
            slot = lax.rem(si, _NBUF_TC)
            nxt = si + _NBUF_TC - 1

            @pl.when(nxt < nslab)
            def _():
                start(nxt, (u + 1) % 2)

            pltpu.make_async_copy(
                x_hbm.at[pl.ds(si * slab_rows, slab_rows)],
                vmem.at[slot],
                sem.at[slot],
            ).wait()
            o_ref[pl.ds(si * slab_rows, slab_rows)] = jnp.sum(
                vmem[slot], axis=(1, 2), keepdims=True
            )


_NC = 2
_NS = 16
_NW = _NC * _NS
_L = 16
_NBUF_SC = 3
_AHEAD = 2
_HCHUNKS = 6


def _sc_pool_body(x_hbm, out_hbm, *refs, plane_base, planes_per_tile,
                  hrows, width):
    bufs = refs[:_NBUF_SC]
    prow = refs[_NBUF_SC]
    sems = refs[_NBUF_SC + 1:]
    wid = lax.axis_index("s") * _NC + lax.axis_index("c")
    plane0 = plane_base + wid * planes_per_tile
    vecs_per_hrow = width // _L

    def start(r, hc, slot):
        pltpu.async_copy(
            x_hbm.at[plane0 + r, pl.ds(hc * hrows, hrows), :],
            bufs[slot],
            sems[slot],
        )

    for g in range(_AHEAD):
        start(0, g, g % _NBUF_SC)

    @pl.loop(0, planes_per_tile)
    def _planes(r):
        accs = [jnp.zeros((_L,), jnp.float32)] * 8
        for hc in range(_HCHUNKS):
            slot = hc % _NBUF_SC
            nslot = (hc + _AHEAD) % _NBUF_SC

            if hc + _AHEAD < _HCHUNKS:

                @pl.when(r < planes_per_tile)
                def _start_same_plane():
                    start(r, hc + _AHEAD, nslot)

            else:

                @pl.when(r + 1 < planes_per_tile)
                def _start_next_plane():
                    start(r + 1, hc + _AHEAD - _HCHUNKS, nslot)

            buf = bufs[slot]
            pltpu.make_async_copy(
                x_hbm.at[plane0, pl.ds(0, hrows), :], buf, sems[slot]
            ).wait()

            @pl.loop(0, hrows, init_carry=tuple(accs))
            def _acc(i, acc):
                acc = list(acc)
                for k in range(vecs_per_hrow):
                    acc[k % 8] = acc[k % 8] + buf[i, pl.ds(k * _L, _L)]
                return tuple(acc)

            accs = list(_acc)

        a = accs
        v = ((a[0] + a[1]) + (a[2] + a[3])) + \
            ((a[4] + a[5]) + (a[6] + a[7]))
        prow[pl.ds(pl.multiple_of(r * _L, _L), _L)] = v

    pltpu.sync_copy(prow, out_hbm.at[pl.ds(wid * planes_per_tile * _L,
                                           planes_per_tile * _L)])


def _proj_body(ptc_ref, psc_ref, mtc_ref, msc_ref, o_ref, *, inv_n):
    s_tc = ptc_ref[...] * inv_n
    s_sc = jnp.sum(psc_ref[...], axis=1, keepdims=True) * inv_n
    o_ref[...] = jax.lax.dot_general(
        mtc_ref[...], s_tc, (((1,), (0,)), ((), ())),
        preferred_element_type=jnp.float32,
    ) + jax.lax.dot_general(
        msc_ref[...], s_sc, (((1,), (0,)), ((), ())),
        preferred_element_type=jnp.float32,
    )


def kernel(x, W):
    B, C, H, Wd = x.shape
    N = H * Wd
    E = W.shape[0]
    R = B * C

    r_sc = 224
    r_tc = R - r_sc

    slab_rows = 8
    nslab = r_tc // slab_rows

    x3 = x.reshape(R, H, Wd)

    tc_sums = pl.pallas_call(
        functools.partial(_tc_pool_body, nslab=nslab, slab_rows=slab_rows),
        in_specs=[pl.BlockSpec(memory_space=pl.ANY)],
        out_specs=pl.BlockSpec(memory_space=pltpu.MemorySpace.VMEM),
        out_shape=jax.ShapeDtypeStruct((r_tc, 1, 1), jnp.float32),
        scratch_shapes=[
            pltpu.VMEM((_NBUF_TC, slab_rows, H, Wd), jnp.float32),
            pltpu.SemaphoreType.DMA((_NBUF_TC,)),
        ],
        compiler_params=pltpu.CompilerParams(
            vmem_limit_bytes=100 * 1024 * 1024,
        ),
    )(x3)

    planes_per_tile = r_sc // _NW
    hrows = H // _HCHUNKS

    mesh = plsc.VectorSubcoreMesh(core_axis_name="c", subcore_axis_name="s")
    sc_partials = pl.kernel(
        functools.partial(
            _sc_pool_body,
            plane_base=r_tc,
            planes_per_tile=planes_per_tile,
            hrows=hrows,
            width=Wd,
        ),
        out_type=jax.ShapeDtypeStruct((r_sc * _L,), jnp.float32),
        mesh=mesh,
        scratch_types=(
            [pltpu.VMEM((hrows, Wd), jnp.float32)] * _NBUF_SC
            + [pltpu.VMEM((planes_per_tile * _L,), jnp.float32)]
            + [pltpu.SemaphoreType.DMA] * _NBUF_SC
        ),
    )(x3)

    M = (jnp.eye(B, dtype=jnp.float32)[:, None, :, None]
         * W[None, :, None, :]).reshape(B * E, R)

    logits_flat = pl.pallas_call(
        functools.partial(_proj_body, inv_n=1.0 / N),
        in_specs=[
            pl.BlockSpec((r_tc, 1), lambda: (0, 0)),
            pl.BlockSpec((r_sc, _L), lambda: (0, 0)),
            pl.BlockSpec((B * E, r_tc), lambda: (0, 0)),
            pl.BlockSpec((B * E, r_sc), lambda: (0, 0)),
        ],
        out_specs=pl.BlockSpec((B * E, 1), lambda: (0, 0)),
        out_shape=jax.ShapeDtypeStruct((B * E, 1), jnp.float32),
    )(tc_sums.reshape(r_tc, 1), sc_partials.reshape(r_sc, _L),
      M[:, :r_tc], M[:, r_tc:])

    return logits_flat.reshape(B, E)


# speedup vs baseline: 1.0318x; 1.0062x over previous
import functools

import jax
import jax.numpy as jnp
from jax import lax
from jax.experimental import pallas as pl
from jax.experimental.pallas import tpu as pltpu
from jax.experimental.pallas import tpu_sc as plsc


_NBUF_TC = 8


def _tc_pool_body(x_hbm, o_ref, vmem, sem, *, nslab, slab_rows):
    def start(si, prio):
        slot = lax.rem(si, _NBUF_TC)
        pltpu.async_copy(
            x_hbm.at[pl.ds(si * slab_rows, slab_rows)],
            vmem.at[slot],
            sem.at[slot],
            priority=prio,
        )

    for s in range(_NBUF_TC - 1):
        start(s, s % 2)

    @pl.loop(0, nslab, step=2)
    def _steps(g):
        for u in range(2):
            si = g + u
            slot = lax.rem(si, _NBUF_TC)
            nxt = si + _NBUF_TC - 1

            @pl.when(nxt < nslab)
            def _():
                start(nxt, (u + 1) % 2)

            pltpu.make_async_copy(
                x_hbm.at[pl.ds(si * slab_rows, slab_rows)],
                vmem.at[slot],
                sem.at[slot],
            ).wait()
            o_ref[pl.ds(si * slab_rows, slab_rows)] = jnp.sum(
                vmem[slot], axis=(1, 2), keepdims=True
            )


_NC = 2
_NS = 16
_NW = _NC * _NS
_L = 16
_NBUF_SC = 3
_AHEAD = 2
_HCHUNKS = 6


def _sc_pool_body(x_hbm, out_hbm, *refs, plane_base, planes_per_tile,
                  hrows, width):
    bufs = refs[:_NBUF_SC]
    prow = refs[_NBUF_SC]
    sems = refs[_NBUF_SC + 1:]
    wid = lax.axis_index("s") * _NC + lax.axis_index("c")
    plane0 = plane_base + wid * planes_per_tile
    vecs_per_hrow = width // _L

    def start(r, hc, slot):
        pltpu.async_copy(
            x_hbm.at[plane0 + r, pl.ds(hc * hrows, hrows), :],
            bufs[slot],
            sems[slot],
        )

    for g in range(_AHEAD):
        start(0, g, g % _NBUF_SC)

    @pl.loop(0, planes_per_tile)
    def _planes(r):
        accs = [jnp.zeros((_L,), jnp.float32)] * 8
        for hc in range(_HCHUNKS):
            slot = hc % _NBUF_SC
            nslot = (hc + _AHEAD) % _NBUF_SC

            if hc + _AHEAD < _HCHUNKS:

                @pl.when(r < planes_per_tile)
                def _start_same_plane():
                    start(r, hc + _AHEAD, nslot)

            else:

                @pl.when(r + 1 < planes_per_tile)
                def _start_next_plane():
                    start(r + 1, hc + _AHEAD - _HCHUNKS, nslot)

            buf = bufs[slot]
            pltpu.make_async_copy(
                x_hbm.at[plane0, pl.ds(0, hrows), :], buf, sems[slot]
            ).wait()

            @pl.loop(0, hrows, init_carry=tuple(accs))
            def _acc(i, acc):
                acc = list(acc)
                for k in range(vecs_per_hrow):
                    acc[k % 8] = acc[k % 8] + buf[i, pl.ds(k * _L, _L)]
                return tuple(acc)

            accs = list(_acc)

        a = accs
        v = ((a[0] + a[1]) + (a[2] + a[3])) + \
            ((a[4] + a[5]) + (a[6] + a[7]))
        prow[pl.ds(pl.multiple_of(r * _L, _L), _L)] = v

    pltpu.sync_copy(prow, out_hbm.at[pl.ds(wid * planes_per_tile * _L,
                                           planes_per_tile * _L)])


def _proj_body(ptc_ref, psc_ref, mtc_ref, msc_ref, o_ref, *, inv_n):
    s_tc = ptc_ref[...] * inv_n
    s_sc = jnp.sum(psc_ref[...], axis=1, keepdims=True) * inv_n
    o_ref[...] = jax.lax.dot_general(
        mtc_ref[...], s_tc, (((1,), (0,)), ((), ())),
        preferred_element_type=jnp.float32,
    ) + jax.lax.dot_general(
        msc_ref[...], s_sc, (((1,), (0,)), ((), ())),
        preferred_element_type=jnp.float32,
    )


def kernel(x, W):
    B, C, H, Wd = x.shape
    N = H * Wd
    E = W.shape[0]
    R = B * C

    r_sc = 192
    r_tc = R - r_sc

    slab_rows = 8
    nslab = r_tc // slab_rows

    x3 = x.reshape(R, H, Wd)

    tc_sums = pl.pallas_call(
        functools.partial(_tc_pool_body, nslab=nslab, slab_rows=slab_rows),
        in_specs=[pl.BlockSpec(memory_space=pl.ANY)],
        out_specs=pl.BlockSpec(memory_space=pltpu.MemorySpace.VMEM),
        out_shape=jax.ShapeDtypeStruct((r_tc, 1, 1), jnp.float32),
        scratch_shapes=[
            pltpu.VMEM((_NBUF_TC, slab_rows, H, Wd), jnp.float32),
            pltpu.SemaphoreType.DMA((_NBUF_TC,)),
        ],
        compiler_params=pltpu.CompilerParams(
            vmem_limit_bytes=100 * 1024 * 1024,
        ),
    )(x3)

    planes_per_tile = r_sc // _NW
    hrows = H // _HCHUNKS

    mesh = plsc.VectorSubcoreMesh(core_axis_name="c", subcore_axis_name="s")
    sc_partials = pl.kernel(
        functools.partial(
            _sc_pool_body,
            plane_base=r_tc,
            planes_per_tile=planes_per_tile,
            hrows=hrows,
            width=Wd,
        ),
        out_type=jax.ShapeDtypeStruct((r_sc * _L,), jnp.float32),
        mesh=mesh,
        scratch_types=(
            [pltpu.VMEM((hrows, Wd), jnp.float32)] * _NBUF_SC
            + [pltpu.VMEM((planes_per_tile * _L,), jnp.float32)]
            + [pltpu.SemaphoreType.DMA] * _NBUF_SC
        ),
    )(x3)

    M = (jnp.eye(B, dtype=jnp.float32)[:, None, :, None]
         * W[None, :, None, :]).reshape(B * E, R)

    logits_flat = pl.pallas_call(
        functools.partial(_proj_body, inv_n=1.0 / N),
        in_specs=[
            pl.BlockSpec((r_tc, 1), lambda: (0, 0)),
            pl.BlockSpec((r_sc, _L), lambda: (0, 0)),
            pl.BlockSpec((B * E, r_tc), lambda: (0, 0)),
            pl.BlockSpec((B * E, r_sc), lambda: (0, 0)),
        ],
        out_specs=pl.BlockSpec((B * E, 1), lambda: (0, 0)),
        out_shape=jax.ShapeDtypeStruct((B * E, 1), jnp.float32),
    )(tc_sums.reshape(r_tc, 1), sc_partials.reshape(r_sc, _L),
      M[:, :r_tc], M[:, r_tc:])

    return logits_flat.reshape(B, E)
